# baseline (speedup 1.0000x reference)
import functools

import jax
import jax.numpy as jnp
from jax import lax
from jax.experimental import pallas as pl
from jax.experimental.pallas import tpu as pltpu
from jax.experimental.pallas import tpu_sc as plsc

N = 10000
E = 320000
D = 128
H = 128
O = 128
EPS = 1e-5

BE = 2000

NC = 2
NS = 16
NW = NC * NS
EW = E // NW
CG = 400

_SC_MESH = plsc.VectorSubcoreMesh(
    core_axis_name="c", subcore_axis_name="s", num_cores=NC, num_subcores=NS)


def _gather_add_body(a_hbm, b_hbm, dst_hbm, src_hbm, p_hbm,
                     idx_d, idx_s, rows, sem):
    wid = lax.axis_index("s") * NC + lax.axis_index("c")
    base = wid * EW

    @pl.loop(0, EW // CG)
    def _chunk(i):
        off = base + i * CG
        pltpu.sync_copy(dst_hbm.at[pl.ds(off, CG)], idx_d)
        pltpu.sync_copy(src_hbm.at[pl.ds(off, CG)], idx_s)
        pltpu.async_copy(a_hbm.at[idx_d], rows, sem).wait()
        pltpu.async_copy(b_hbm.at[idx_s], rows, sem, add=True).wait()
        pltpu.sync_copy(rows, p_hbm.at[pl.ds(off, CG)])


FG = 8
EH = E // 2
SCC = 1600
NCHUNK = EH // SCC
GPC = SCC // 16
PC = 2000


def _rot8(v):
    perm8 = (lax.iota(jnp.int32, 16) + 8) & 15
    return v.at[perm8].get(mode="promise_in_bounds")


def _scatter_max_body(h_hbm, dst_hbm, out_hbm, xch_hbm,
                      m_v, hb0, hb1, ib0, ib1, pbuf,
                      semh0, semh1, semi0, semi1):
    c = lax.axis_index("c")
    s = lax.axis_index("s")
    g = s >> 1
    gcol = (c * 8 + g) * FG
    q = s & 1
    iota = lax.iota(jnp.int32, 16)
    rowpat = iota >> 3
    colpat = iota & 7
    ge8 = iota >= 8
    perm8 = (iota + 8) & 15
    colvs = [(iota + r) & 7 for r in range(FG)]
    neg = jnp.full((16,), -jnp.inf, jnp.float32)
    hbufs = (hb0, hb1)
    ibufs = (ib0, ib1)
    semhs = (semh0, semh1)
    semis = (semi0, semi1)

    @pl.loop(0, N // 2)
    def _init(j):
        plsc.store_scatter(m_v, [j * 2 + rowpat, colpat], neg)

    def _issue(cidx, b):
        off = q * EH + cidx * SCC
        pltpu.async_copy(dst_hbm.at[pl.ds(off, SCC)], ibufs[b], semis[b])
        pltpu.async_copy(h_hbm.at[pl.ds(off, SCC), pl.ds(gcol, FG)],
                         hbufs[b], semhs[b])

    _issue(0, 0)
    _issue(1, 1)

    @pl.loop(0, NCHUNK // 2)
    def _chunk(i):
        for b in range(2):
            cidx = i * 2 + b
            off = q * EH + cidx * SCC
            ib = ibufs[b]
            hb = hbufs[b]
            pltpu.make_async_copy(dst_hbm.at[pl.ds(off, SCC)], ib,
                                  semis[b]).wait()
            pltpu.make_async_copy(h_hbm.at[pl.ds(off, SCC), pl.ds(gcol, FG)],
                                  hb, semhs[b]).wait()

            @pl.loop(0, GPC)
            def _group(j):
                dst16 = ib[pl.ds(j * 16, 16)]
                rowv = j * 16 + iota
                rowr = j * 16 + perm8
                dstr = plsc.load_gather(ib, [rowr])
                eq = dst16 == dstr
                smask = jnp.logical_not(jnp.logical_and(eq, ge8))
                dup = eq & False
                for k in range(1, 8):
                    dk = plsc.load_gather(ib, [j * 16 + ((iota + k) & 15)])
                    dup = jnp.logical_or(dup, dst16 == dk)
                anydup = jnp.any(dup)
                hvs = [plsc.load_gather(hb, [rowv, colvs[r]])
                       for r in range(FG)]
                hrs = [plsc.load_gather(hb, [rowr, colvs[r]])
                       for r in range(FG)]
                hv2s = [jnp.where(eq, jnp.maximum(hvs[r], hrs[r]), hvs[r])
                        for r in range(FG)]

                @pl.when(jnp.logical_not(anydup))
                def _fast():
                    curs = [plsc.load_gather(m_v, [dst16, colvs[r]])
                            for r in range(FG)]
                    for r in range(FG):
                        plsc.store_scatter(m_v, [dst16, colvs[r]],
                                           jnp.maximum(curs[r], hv2s[r]),
                                           mask=smask)

                @pl.when(anydup)
                def _slow():
                    for r in range(FG):
                        cur = plsc.load_gather(m_v, [dst16, colvs[r]])
                        plsc.store_scatter(m_v, [dst16, colvs[r]],
                                           jnp.maximum(cur, hv2s[r]),
                                           mask=smask)

            nxt = cidx + 2

            @pl.when(nxt < NCHUNK)
            def _prefetch():
                _issue(nxt, b)

    gg = c * 8 + g

    @pl.when(q == 1)
    def _publish():
        pltpu.sync_copy(m_v, xch_hbm.at[gg])

    plsc.subcore_barrier()

    @pl.when(q == 0)
    def _merge():
        @pl.loop(0, N // PC)
        def _mch(k):
            pltpu.sync_copy(xch_hbm.at[gg, pl.ds(k * PC, PC)], pbuf)

            @pl.loop(0, PC // 2)
            def _mrow(j):
                rv = j * 2 + rowpat
                rvm = k * PC + j * 2 + rowpat
                mv = plsc.load_gather(m_v, [rvm, colpat])
                pv = plsc.load_gather(pbuf, [rv, colpat])
                mx = jnp.maximum(jnp.maximum(mv, pv), 0.0)
                plsc.store_scatter(m_v, [rvm, colpat], mx)

        pltpu.sync_copy(m_v, out_hbm.at[:, pl.ds(gcol, FG)])


_scatter_max = functools.partial(
    pl.kernel,
    out_type=(
        jax.ShapeDtypeStruct((N, O), jnp.float32),
        jax.ShapeDtypeStruct((16, N, FG), jnp.float32),
    ),
    mesh=_SC_MESH,
    scratch_types=[
        pltpu.VMEM((N, FG), jnp.float32),
        pltpu.VMEM((SCC, FG), jnp.float32),
        pltpu.VMEM((SCC, FG), jnp.float32),
        pltpu.VMEM((SCC,), jnp.int32),
        pltpu.VMEM((SCC,), jnp.int32),
        pltpu.VMEM((PC, FG), jnp.float32),
        pltpu.SemaphoreType.DMA,
        pltpu.SemaphoreType.DMA,
        pltpu.SemaphoreType.DMA,
        pltpu.SemaphoreType.DMA,
    ],
    compiler_params=pltpu.CompilerParams(
        use_tc_tiling_on_sc=False, needs_layout_passes=False),
)(_scatter_max_body)


_gather_add = functools.partial(
    pl.kernel,
    out_type=jax.ShapeDtypeStruct((E, H), jnp.float32),
    mesh=_SC_MESH,
    scratch_types=[
        pltpu.VMEM((CG,), jnp.int32),
        pltpu.VMEM((CG,), jnp.int32),
        pltpu.VMEM((CG, H), jnp.float32),
        pltpu.SemaphoreType.DMA,
    ],
)(_gather_add_body)


def _node_tables_kernel(x_ref, gamma_ref, beta_ref, w1d_ref, w1b_ref, b1_ref,
                        a_ref, b_ref):
    x = x_ref[...]
    mean = jnp.mean(x, axis=0, keepdims=True)
    var = jnp.mean((x - mean) ** 2, axis=0, keepdims=True)
    scale = gamma_ref[...] * jax.lax.rsqrt(var + EPS)
    xn = (x - mean) * scale + beta_ref[...]
    a_ref[...] = jnp.dot(xn, w1d_ref[...], preferred_element_type=jnp.float32) + b1_ref[...]
    b_ref[...] = jnp.dot(xn, w1b_ref[...], preferred_element_type=jnp.float32)


def _edge_mlp_kernel(p_ref, w2_ref, b2_ref, h_ref):
    p = jnp.maximum(p_ref[...], 0.0)
    h_ref[...] = jnp.dot(p, w2_ref[...], preferred_element_type=jnp.float32) + b2_ref[...]


def kernel(x, edge_index, gamma, beta, W1, b1, W2, b2):
    w1d = W1[:D] - W1[D:]
    w1b = W1[D:]
    a_tab, b_tab = pl.pallas_call(
        _node_tables_kernel,
        out_shape=(
            jax.ShapeDtypeStruct((N, H), jnp.float32),
            jax.ShapeDtypeStruct((N, H), jnp.float32),
        ),
    )(x, gamma.reshape(1, D), beta.reshape(1, D), w1d, w1b, b1.reshape(1, H))

    src = edge_index[0]
    dst = edge_index[1]
    p = _gather_add(a_tab, b_tab, dst, src)

    h = pl.pallas_call(
        _edge_mlp_kernel,
        grid=(E // BE,),
        in_specs=[
            pl.BlockSpec((BE, H), lambda i: (i, 0)),
            pl.BlockSpec((H, O), lambda i: (0, 0)),
            pl.BlockSpec((1, O), lambda i: (0, 0)),
        ],
        out_specs=pl.BlockSpec((BE, O), lambda i: (i, 0)),
        out_shape=jax.ShapeDtypeStruct((E, O), jnp.float32),
    )(p, W2, b2.reshape(1, O))

    out, _ = _scatter_max(h, dst)
    return out


# speedup vs baseline: 1.2570x; 1.2570x over previous
import functools

import jax
import jax.numpy as jnp
from jax import lax
from jax.experimental import pallas as pl
from jax.experimental.pallas import tpu as pltpu
from jax.experimental.pallas import tpu_sc as plsc

N = 10000
E = 320000
D = 128
H = 128
O = 128
EPS = 1e-5

BE = 2000

NC = 2
NS = 16
NW = NC * NS
EW = E // NW
CG = 400

_SC_MESH = plsc.VectorSubcoreMesh(
    core_axis_name="c", subcore_axis_name="s", num_cores=NC, num_subcores=NS)


def _gather_add_body(a_hbm, b_hbm, dst_hbm, src_hbm, p_hbm,
                     idx_d, idx_s, rows, sem):
    wid = lax.axis_index("s") * NC + lax.axis_index("c")
    base = wid * EW

    @pl.loop(0, EW // CG)
    def _chunk(i):
        off = base + i * CG
        pltpu.sync_copy(dst_hbm.at[pl.ds(off, CG)], idx_d)
        pltpu.sync_copy(src_hbm.at[pl.ds(off, CG)], idx_s)
        pltpu.async_copy(a_hbm.at[idx_d], rows, sem).wait()
        pltpu.async_copy(b_hbm.at[idx_s], rows, sem, add=True).wait()
        pltpu.sync_copy(rows, p_hbm.at[pl.ds(off, CG)])


FG = 8
NQ = 4
EQ = E // NQ
SCC = 800
NCHUNK = EQ // SCC
GPC = SCC // 16
PCW = 16000
NEP = 1000
MINF2 = -8323200


def _bf16max(a_i32, b_i32):
    a = plsc.bitcast(a_i32, jnp.bfloat16)
    b = plsc.bitcast(b_i32, jnp.bfloat16)
    return plsc.bitcast(jnp.maximum(a, b), jnp.int32)


def _scatter_max_body(h_hbm, dst_hbm, out_hbm, xch_hbm,
                      m_v, hb0, hb1, ib0, ib1, pbuf, olo, ohi,
                      semh0, semh1, semi0, semi1):
    c = lax.axis_index("c")
    s = lax.axis_index("s")
    g = s >> 2
    gg = c * 4 + g
    gcol = gg * FG
    q = s & 3
    wid = c * NS + s
    iota = lax.iota(jnp.int32, 16)
    rowpat = iota >> 3
    colpat = iota & 7
    ge8 = iota >= 8
    perm8 = (iota + 8) & 15
    iota8 = iota * 8
    perm8x8 = perm8 * 8
    colvs = [(iota + r) & 7 for r in range(FG)]
    minf = jnp.full((16,), MINF2, jnp.int32)
    hbufs = (hb0, hb1)
    ibufs = (ib0, ib1)
    semhs = (semh0, semh1)
    semis = (semi0, semi1)

    @pl.loop(0, N * FG // 16)
    def _init(j):
        m_v[pl.ds(j * 16, 16)] = minf

    def _issue(cidx, b):
        off = q * EQ + cidx * SCC
        pltpu.async_copy(dst_hbm.at[pl.ds(off, SCC)], ibufs[b], semis[b])
        pltpu.async_copy(h_hbm.at[pl.ds(off, SCC), pl.ds(gcol, FG)],
                         hbufs[b], semhs[b])

    _issue(0, 0)
    _issue(1, 1)

    @pl.loop(0, NCHUNK // 2)
    def _chunk(i):
        for b in range(2):
            cidx = i * 2 + b
            off = q * EQ + cidx * SCC
            ib = ibufs[b]
            hb = hbufs[b]
            pltpu.make_async_copy(dst_hbm.at[pl.ds(off, SCC)], ib,
                                  semis[b]).wait()
            pltpu.make_async_copy(h_hbm.at[pl.ds(off, SCC), pl.ds(gcol, FG)],
                                  hb, semhs[b]).wait()

            @pl.loop(0, GPC)
            def _group(j):
                dst16 = ib[pl.ds(j * 16, 16)]
                dst8 = dst16 * 8
                rowv = j * 16 + iota
                rowr = j * 16 + perm8
                dstr = plsc.load_gather(ib, [rowr])
                eq = dst16 == dstr
                smask = jnp.logical_not(jnp.logical_and(eq, ge8))
                hvs = [plsc.load_gather(hb, [rowv, colvs[r]])
                       for r in range(FG)]
                hrs = [plsc.load_gather(hb, [rowr, colvs[r]])
                       for r in range(FG)]
                for r in range(FG):
                    hv2 = jnp.where(eq, _bf16max(hvs[r], hrs[r]), hvs[r])
                    cur = plsc.load_gather(m_v, [dst8 + colvs[r]])
                    plsc.store_scatter(m_v, [dst8 + colvs[r]],
                                       _bf16max(cur, hv2), mask=smask)

            nxt = cidx + 2

            @pl.when(nxt < NCHUNK)
            def _prefetch():
                _issue(nxt, b)

    @pl.when(q != 0)
    def _publish():
        pltpu.sync_copy(m_v, xch_hbm.at[wid])

    plsc.subcore_barrier()

    @pl.when(q == 0)
    def _merge():
        for t in (1, 2, 3):
            @pl.loop(0, N * FG // PCW)
            def _mch(k):
                pltpu.sync_copy(xch_hbm.at[wid + t, pl.ds(k * PCW, PCW)],
                                pbuf)

                @pl.loop(0, PCW // 16)
                def _mvec(j):
                    idx = k * PCW + j * 16
                    m_v[pl.ds(idx, 16)] = _bf16max(m_v[pl.ds(idx, 16)],
                                                   pbuf[pl.ds(j * 16, 16)])

        @pl.loop(0, N // NEP)
        def _ep(k):
            @pl.loop(0, NEP * FG // 16)
            def _ev(j):
                w = m_v[pl.ds(k * NEP * FG + j * 16, 16)]
                lo = jnp.maximum(
                    plsc.bitcast(jnp.left_shift(w, 16), jnp.float32), 0.0)
                hi = jnp.maximum(
                    plsc.bitcast(w & jnp.int32(-65536), jnp.float32), 0.0)
                plsc.store_scatter(olo, [j * 2 + rowpat, colpat], lo)
                plsc.store_scatter(ohi, [j * 2 + rowpat, colpat], hi)

            pltpu.sync_copy(olo, out_hbm.at[pl.ds(k * NEP, NEP),
                                            pl.ds(gcol, FG)])
            pltpu.sync_copy(ohi, out_hbm.at[pl.ds(k * NEP, NEP),
                                            pl.ds(64 + gcol, FG)])


_scatter_max = functools.partial(
    pl.kernel,
    out_type=(
        jax.ShapeDtypeStruct((N, O), jnp.float32),
        jax.ShapeDtypeStruct((NW, N * FG), jnp.int32),
    ),
    mesh=_SC_MESH,
    scratch_types=[
        pltpu.VMEM((N * FG,), jnp.int32),
        pltpu.VMEM((SCC, FG), jnp.int32),
        pltpu.VMEM((SCC, FG), jnp.int32),
        pltpu.VMEM((SCC,), jnp.int32),
        pltpu.VMEM((SCC,), jnp.int32),
        pltpu.VMEM((PCW,), jnp.int32),
        pltpu.VMEM((NEP, FG), jnp.float32),
        pltpu.VMEM((NEP, FG), jnp.float32),
        pltpu.SemaphoreType.DMA,
        pltpu.SemaphoreType.DMA,
        pltpu.SemaphoreType.DMA,
        pltpu.SemaphoreType.DMA,
    ],
    compiler_params=pltpu.CompilerParams(
        use_tc_tiling_on_sc=False, needs_layout_passes=False),
)(_scatter_max_body)


_gather_add = functools.partial(
    pl.kernel,
    out_type=jax.ShapeDtypeStruct((E, H), jnp.float32),
    mesh=_SC_MESH,
    scratch_types=[
        pltpu.VMEM((CG,), jnp.int32),
        pltpu.VMEM((CG,), jnp.int32),
        pltpu.VMEM((CG, H), jnp.float32),
        pltpu.SemaphoreType.DMA,
    ],
)(_gather_add_body)


def _node_tables_kernel(x_ref, gamma_ref, beta_ref, w1d_ref, w1b_ref, b1_ref,
                        a_ref, b_ref):
    x = x_ref[...]
    mean = jnp.mean(x, axis=0, keepdims=True)
    var = jnp.mean((x - mean) ** 2, axis=0, keepdims=True)
    scale = gamma_ref[...] * jax.lax.rsqrt(var + EPS)
    xn = (x - mean) * scale + beta_ref[...]
    a_ref[...] = jnp.dot(xn, w1d_ref[...], preferred_element_type=jnp.float32) + b1_ref[...]
    b_ref[...] = jnp.dot(xn, w1b_ref[...], preferred_element_type=jnp.float32)


def _edge_mlp_kernel(p_ref, w2_ref, b2_ref, h_ref):
    p = jnp.maximum(p_ref[...], 0.0)
    h = jnp.dot(p, w2_ref[...], preferred_element_type=jnp.float32) + b2_ref[...]
    lo = lax.bitcast_convert_type(
        h[:, :64].astype(jnp.bfloat16), jnp.uint16).astype(jnp.uint32)
    hi = lax.bitcast_convert_type(
        h[:, 64:].astype(jnp.bfloat16), jnp.uint16).astype(jnp.uint32)
    h_ref[...] = lax.bitcast_convert_type(lo | (hi << 16), jnp.int32)


def kernel(x, edge_index, gamma, beta, W1, b1, W2, b2):
    w1d = W1[:D] - W1[D:]
    w1b = W1[D:]
    a_tab, b_tab = pl.pallas_call(
        _node_tables_kernel,
        out_shape=(
            jax.ShapeDtypeStruct((N, H), jnp.float32),
            jax.ShapeDtypeStruct((N, H), jnp.float32),
        ),
    )(x, gamma.reshape(1, D), beta.reshape(1, D), w1d, w1b, b1.reshape(1, H))

    src = edge_index[0]
    dst = edge_index[1]
    p = _gather_add(a_tab, b_tab, dst, src)

    h = pl.pallas_call(
        _edge_mlp_kernel,
        grid=(E // BE,),
        in_specs=[
            pl.BlockSpec((BE, H), lambda i: (i, 0)),
            pl.BlockSpec((H, O), lambda i: (0, 0)),
            pl.BlockSpec((1, O), lambda i: (0, 0)),
        ],
        out_specs=pl.BlockSpec((BE, O // 2), lambda i: (i, 0)),
        out_shape=jax.ShapeDtypeStruct((E, O // 2), jnp.int32),
    )(p, W2, b2.reshape(1, O))

    out, _ = _scatter_max(h, dst)
    return out
